# baseline (speedup 1.0000x reference)
import functools

import jax
import jax.numpy as jnp
from jax import lax
from jax.experimental import pallas as pl
from jax.experimental.pallas import tpu as pltpu
from jax.experimental.pallas import tpu_sc as plsc

_N = 10000
_D = 128
_NPAD = 10240
_NC = 2
_NS = 16
_W = _NC * _NS
_RPT = _NPAD // _NS
_CK = 64
_CW = 160
_GC = 80
_NBUF = 4
_DCK = 128
_DCW = 80
_EPAD = _W * _CW * _CK
_DPAD = _EPAD
_BN = 1024

_mesh = plsc.VectorSubcoreMesh(core_axis_name="c", subcore_axis_name="s")


def _sc_deg_body(dst2d, deg0, deg1, idx_v, ones_v, zero_v, deg_sh, sem):
    c = lax.axis_index("c")
    s = lax.axis_index("s")
    wid = c * _NS + s

    def fill_ones(i, carry):
        ones_v[pl.ds(i * 16, 16)] = jnp.full((16,), 1.0, jnp.float32)
        return carry

    lax.fori_loop(0, _DCK // 16, fill_ones, 0)

    def fill_zero(i, carry):
        zero_v[pl.ds(i * 16, 16)] = jnp.zeros((16,), jnp.float32)
        return carry

    lax.fori_loop(0, 640 // 16, fill_zero, 0)

    pltpu.sync_copy(zero_v.at[pl.ds(0, _RPT)], deg_sh.at[pl.ds(s * _RPT, _RPT)])
    pltpu.sync_copy(dst2d.at[pl.ds(wid * _DCW, _DCW)], idx_v)
    plsc.subcore_barrier()

    def body(j, carry):
        pltpu.sync_copy(ones_v, deg_sh.at[idx_v.at[j]], add=True)
        return carry

    lax.fori_loop(0, _DCW, body, 0)
    plsc.subcore_barrier()

    pltpu.sync_copy(deg_sh.at[pl.ds(s * _RPT, _RPT)], zero_v.at[pl.ds(0, _RPT)])

    @pl.when(c == 0)
    def _():
        pltpu.sync_copy(zero_v.at[pl.ds(0, _RPT)], deg0.at[pl.ds(s * _RPT, _RPT)])

    @pl.when(c == 1)
    def _():
        pltpu.sync_copy(zero_v.at[pl.ds(0, _RPT)], deg1.at[pl.ds(s * _RPT, _RPT)])


@functools.partial(
    pl.kernel,
    out_type=(
        jax.ShapeDtypeStruct((_NPAD,), jnp.float32),
        jax.ShapeDtypeStruct((_NPAD,), jnp.float32),
    ),
    mesh=_mesh,
    scratch_types=[
        pltpu.VMEM((_DCW, _DCK), jnp.int32),
        pltpu.VMEM((_DCK,), jnp.float32),
        pltpu.VMEM((640,), jnp.float32),
        pltpu.VMEM_SHARED((_NPAD,), jnp.float32),
        pltpu.SemaphoreType.DMA,
    ],
)
def _sc_deg(*refs):
    _sc_deg_body(*refs)


def _sc_seg_body(
    g_hbm, src3d, dst3d, out0, out1, sidx, didx, rows4, acc_sh,
    gsem0, gsem1, gsem2, gsem3, ssem0, ssem1, ssem2, ssem3,
):
    c = lax.axis_index("c")
    s = lax.axis_index("s")
    wid = c * _NS + s
    bufs = [rows4.at[b] for b in range(_NBUF)]
    gsem = [gsem0, gsem1, gsem2, gsem3]
    ssem = [ssem0, ssem1, ssem2, ssem3]
    dummy = g_hbm.at[pl.ds(0, _CK)]
    niter = _GC // _NBUF

    def fz(i, carry):
        rows4[0, i // 8, pl.ds((i % 8) * 16, 16)] = jnp.zeros((16,), jnp.float32)
        return carry

    lax.fori_loop(0, _CK * 8, fz, 0)
    for q in range(_RPT // _CK):
        pltpu.sync_copy(bufs[0], acc_sh.at[pl.ds(s * _RPT + q * _CK, _CK)])
    plsc.subcore_barrier()

    def src_chunk(qq_half, odd):
        return sidx.at[qq_half, pl.ds(64 * odd, _CK)]

    for g in range(_CW // _GC):
        if g > 0:
            for b in range(_NBUF):
                pltpu.make_async_copy(bufs[b], acc_sh.at[didx.at[0]], ssem[b]).wait()
        pltpu.sync_copy(src3d.at[wid * (_CW // _GC) + g], sidx)
        pltpu.sync_copy(dst3d.at[wid * (_CW // _GC) + g], didx)
        for b in range(_NBUF - 1):
            pltpu.async_copy(g_hbm.at[src_chunk(b // 2, b % 2)], bufs[b], gsem[b])

        def body(t0, carry):
            for i in range(_NBUF):
                q = _NBUF * t0 + i
                j = (i + _NBUF - 1) % _NBUF
                row = (i + 3) // 2
                odd = (i + 3) % 2
                pltpu.make_async_copy(dummy, bufs[i], gsem[i]).wait()
                pltpu.async_copy(bufs[i], acc_sh.at[didx.at[q]], ssem[i], add=True)
                if i == 0:
                    @pl.when(t0 > 0)
                    def _():
                        pltpu.make_async_copy(bufs[j], acc_sh.at[didx.at[0]], ssem[j]).wait()

                    pltpu.async_copy(g_hbm.at[src_chunk(2 * t0 + row, odd)], bufs[j], gsem[j])
                else:
                    @pl.when(t0 < niter - 1)
                    def _():
                        pltpu.make_async_copy(bufs[j], acc_sh.at[didx.at[0]], ssem[j]).wait()
                        pltpu.async_copy(g_hbm.at[src_chunk(2 * t0 + row, odd)], bufs[j], gsem[j])

            return carry

        lax.fori_loop(0, niter, body, 0)

    for b in range(_NBUF):
        pltpu.make_async_copy(bufs[b], acc_sh.at[didx.at[0]], ssem[b]).wait()
    plsc.subcore_barrier()

    @pl.when(c == 0)
    def _():
        pltpu.sync_copy(acc_sh.at[pl.ds(s * _RPT, _RPT)], out0.at[pl.ds(s * _RPT, _RPT)])

    @pl.when(c == 1)
    def _():
        pltpu.sync_copy(acc_sh.at[pl.ds(s * _RPT, _RPT)], out1.at[pl.ds(s * _RPT, _RPT)])


@functools.partial(
    pl.kernel,
    out_type=(
        jax.ShapeDtypeStruct((_NPAD, _D), jnp.float32),
        jax.ShapeDtypeStruct((_NPAD, _D), jnp.float32),
    ),
    mesh=_mesh,
    scratch_types=[
        pltpu.VMEM((_GC // 2, 2 * _CK), jnp.int32),
        pltpu.VMEM((_GC, _CK), jnp.int32),
        pltpu.VMEM((_NBUF, _CK, _D), jnp.float32),
        pltpu.VMEM_SHARED((_NPAD, _D), jnp.float32),
        pltpu.SemaphoreType.DMA,
        pltpu.SemaphoreType.DMA,
        pltpu.SemaphoreType.DMA,
        pltpu.SemaphoreType.DMA,
        pltpu.SemaphoreType.DMA,
        pltpu.SemaphoreType.DMA,
        pltpu.SemaphoreType.DMA,
        pltpu.SemaphoreType.DMA,
    ],
)
def _sc_seg(*refs):
    _sc_seg_body(*refs)


def _tc_prep_body(deg0_ref, deg1_ref, x_ref, dinv_ref, g_ref):
    d = deg0_ref[...] + deg1_ref[...] + 1.0
    dcol = lax.rsqrt(d)[:, None]
    dinv_ref[...] = dcol
    g_ref[...] = dcol * x_ref[...]


def _tc_prep(deg0, deg1, x_pad):
    grid = _NPAD // _BN
    return pl.pallas_call(
        _tc_prep_body,
        grid=(grid,),
        in_specs=[
            pl.BlockSpec((_BN,), lambda i: (i,)),
            pl.BlockSpec((_BN,), lambda i: (i,)),
            pl.BlockSpec((_BN, _D), lambda i: (i, 0)),
        ],
        out_specs=[
            pl.BlockSpec((_BN, 1), lambda i: (i, 0)),
            pl.BlockSpec((_BN, _D), lambda i: (i, 0)),
        ],
        out_shape=[
            jax.ShapeDtypeStruct((_NPAD, 1), jnp.float32),
            jax.ShapeDtypeStruct((_NPAD, _D), jnp.float32),
        ],
    )(deg0, deg1, x_pad)


def _tc_mid_body(acc0_ref, acc1_ref, g_ref, dinv_ref, w_ref, b_ref, out_ref):
    dcol = dinv_ref[...]
    agg = dcol * (acc0_ref[...] + acc1_ref[...] + g_ref[...])
    h = jnp.dot(agg, w_ref[...], preferred_element_type=jnp.float32) + b_ref[...]
    out_ref[...] = dcol * jnp.maximum(h, 0.0)


def _tc_mid(acc0, acc1, g, dinv, w1t, b1r):
    grid = _NPAD // _BN
    return pl.pallas_call(
        _tc_mid_body,
        grid=(grid,),
        in_specs=[
            pl.BlockSpec((_BN, _D), lambda i: (i, 0)),
            pl.BlockSpec((_BN, _D), lambda i: (i, 0)),
            pl.BlockSpec((_BN, _D), lambda i: (i, 0)),
            pl.BlockSpec((_BN, 1), lambda i: (i, 0)),
            pl.BlockSpec((_D, _D), lambda i: (0, 0)),
            pl.BlockSpec((1, _D), lambda i: (0, 0)),
        ],
        out_specs=pl.BlockSpec((_BN, _D), lambda i: (i, 0)),
        out_shape=jax.ShapeDtypeStruct((_NPAD, _D), jnp.float32),
    )(acc0, acc1, g, dinv, w1t, b1r)


def _tc_final_body(acc0_ref, acc1_ref, g_ref, dinv_ref, w2_ref, b2_ref, w3_ref, b3_ref, out_ref):
    agg = dinv_ref[...] * (acc0_ref[...] + acc1_ref[...] + g_ref[...])
    h = jnp.dot(agg, w2_ref[...], preferred_element_type=jnp.float32) + b2_ref[...]
    h = jnp.maximum(h, 0.0)
    out_ref[...] = jnp.dot(h, w3_ref[...], preferred_element_type=jnp.float32) + b3_ref[...]


def _tc_final(acc0, acc1, g, dinv, w2t, b2r, w3t, b3r):
    grid = _NPAD // _BN
    cpad = w3t.shape[1]
    return pl.pallas_call(
        _tc_final_body,
        grid=(grid,),
        in_specs=[
            pl.BlockSpec((_BN, _D), lambda i: (i, 0)),
            pl.BlockSpec((_BN, _D), lambda i: (i, 0)),
            pl.BlockSpec((_BN, _D), lambda i: (i, 0)),
            pl.BlockSpec((_BN, 1), lambda i: (i, 0)),
            pl.BlockSpec((_D, _D), lambda i: (0, 0)),
            pl.BlockSpec((1, _D), lambda i: (0, 0)),
            pl.BlockSpec((_D, cpad), lambda i: (0, 0)),
            pl.BlockSpec((1, cpad), lambda i: (0, 0)),
        ],
        out_specs=pl.BlockSpec((_BN, cpad), lambda i: (i, 0)),
        out_shape=jax.ShapeDtypeStruct((_NPAD, cpad), jnp.float32),
    )(acc0, acc1, g, dinv, w2t, b2r, w3t, b3r)


def kernel(x, edge_index, W1, b1, W2, b2, W3, b3):
    n, d = x.shape
    e = edge_index.shape[1]
    cpad = 48

    x_pad = jnp.zeros((_NPAD, _D), jnp.float32).at[:n].set(x)
    src = jnp.full((_EPAD,), _N, jnp.int32).at[:e].set(edge_index[0])
    dst = jnp.full((_EPAD,), _N, jnp.int32).at[:e].set(edge_index[1])
    ngrp = _CW // _GC
    src2d = src.reshape(_W * ngrp, _GC // 2, 2 * _CK)
    dst2d = dst.reshape(_W * ngrp, _GC, _CK)
    dst2d_deg = dst[:_DPAD].reshape(_W * _DCW, _DCK)

    w1t = W1.T
    w2t = W2.T
    w3t = jnp.zeros((_D, cpad), jnp.float32).at[:, : W3.shape[0]].set(W3.T)
    b1r = b1.reshape(1, _D)
    b2r = b2.reshape(1, _D)
    b3r = jnp.zeros((1, cpad), jnp.float32).at[0, : W3.shape[0]].set(b3)

    deg0, deg1 = _sc_deg(dst2d_deg)
    dinv, g1 = _tc_prep(deg0, deg1, x_pad)
    a10, a11 = _sc_seg(g1, src2d, dst2d)
    g2 = _tc_mid(a10, a11, g1, dinv, w1t, b1r)
    a20, a21 = _sc_seg(g2, src2d, dst2d)
    outp = _tc_final(a20, a21, g2, dinv, w2t, b2r, w3t, b3r)
    return outp[:n, : W3.shape[0]]


# speedup vs baseline: 1.1919x; 1.1919x over previous
import functools

import jax
import jax.numpy as jnp
from jax import lax
from jax.experimental import pallas as pl
from jax.experimental.pallas import tpu as pltpu
from jax.experimental.pallas import tpu_sc as plsc

_N = 10000
_D = 128
_NPAD = 10240
_NC = 2
_NS = 16
_W = _NC * _NS
_RPT = _NPAD // _NS
_CK = 128
_CW = 80
_GC = 16
_EPAD = _W * _CW * _CK
_BN = 1024

_mesh = plsc.VectorSubcoreMesh(core_axis_name="c", subcore_axis_name="s")


def _sc_deg_body(dst2d, deg0, deg1, idx_v, ones_v, zero_v, deg_sh, sem):
    c = lax.axis_index("c")
    s = lax.axis_index("s")
    wid = c * _NS + s

    def fill_ones(i, carry):
        ones_v[pl.ds(i * 16, 16)] = jnp.full((16,), 1.0, jnp.float32)
        return carry

    lax.fori_loop(0, _CK // 16, fill_ones, 0)

    def fill_zero(i, carry):
        zero_v[pl.ds(i * 16, 16)] = jnp.zeros((16,), jnp.float32)
        return carry

    lax.fori_loop(0, 640 // 16, fill_zero, 0)

    pltpu.sync_copy(zero_v.at[pl.ds(0, _RPT)], deg_sh.at[pl.ds(s * _RPT, _RPT)])
    pltpu.sync_copy(dst2d.at[pl.ds(wid * _CW, _CW)], idx_v)
    plsc.subcore_barrier()

    def body(j, carry):
        pltpu.sync_copy(ones_v, deg_sh.at[idx_v.at[j]], add=True)
        return carry

    lax.fori_loop(0, _CW, body, 0)
    plsc.subcore_barrier()

    pltpu.sync_copy(deg_sh.at[pl.ds(s * _RPT, _RPT)], zero_v.at[pl.ds(0, _RPT)])

    @pl.when(c == 0)
    def _():
        pltpu.sync_copy(zero_v.at[pl.ds(0, _RPT)], deg0.at[pl.ds(s * _RPT, _RPT)])

    @pl.when(c == 1)
    def _():
        pltpu.sync_copy(zero_v.at[pl.ds(0, _RPT)], deg1.at[pl.ds(s * _RPT, _RPT)])


@functools.partial(
    pl.kernel,
    out_type=(
        jax.ShapeDtypeStruct((_NPAD,), jnp.float32),
        jax.ShapeDtypeStruct((_NPAD,), jnp.float32),
    ),
    mesh=_mesh,
    scratch_types=[
        pltpu.VMEM((_CW, _CK), jnp.int32),
        pltpu.VMEM((_CK,), jnp.float32),
        pltpu.VMEM((640,), jnp.float32),
        pltpu.VMEM_SHARED((_NPAD,), jnp.float32),
        pltpu.SemaphoreType.DMA,
    ],
)
def _sc_deg(*refs):
    _sc_deg_body(*refs)


def _sc_seg_body(g_hbm, src2d, dst2d, out0, out1, sidx, didx, rows2, acc_sh, sem0, sem1):
    c = lax.axis_index("c")
    s = lax.axis_index("s")
    wid = c * _NS + s
    buf0 = rows2.at[0]
    buf1 = rows2.at[1]
    dummy = g_hbm.at[pl.ds(0, _CK)]

    def fz(i, carry):
        rows2[0, i // 8, pl.ds((i % 8) * 16, 16)] = jnp.zeros((16,), jnp.float32)
        return carry

    lax.fori_loop(0, _CK * 8, fz, 0)
    for q in range(_RPT // _CK):
        pltpu.sync_copy(buf0, acc_sh.at[pl.ds(s * _RPT + q * _CK, _CK)])
    plsc.subcore_barrier()

    for g in range(_CW // _GC):
        base = wid * _CW + g * _GC
        pltpu.sync_copy(src2d.at[pl.ds(base, _GC)], sidx)
        pltpu.sync_copy(dst2d.at[pl.ds(base, _GC)], didx)
        pltpu.async_copy(g_hbm.at[sidx.at[0]], buf0, sem0)

        def body(t, carry):
            pltpu.async_copy(g_hbm.at[sidx.at[2 * t + 1]], buf1, sem1)
            pltpu.make_async_copy(dummy, buf0, sem0).wait()
            pltpu.sync_copy(buf0, acc_sh.at[didx.at[2 * t]], add=True)

            @pl.when(t < _GC // 2 - 1)
            def _():
                pltpu.async_copy(g_hbm.at[sidx.at[2 * t + 2]], buf0, sem0)

            pltpu.make_async_copy(dummy, buf1, sem1).wait()
            pltpu.sync_copy(buf1, acc_sh.at[didx.at[2 * t + 1]], add=True)
            return carry

        lax.fori_loop(0, _GC // 2, body, 0)
    plsc.subcore_barrier()

    @pl.when(c == 0)
    def _():
        pltpu.sync_copy(acc_sh.at[pl.ds(s * _RPT, _RPT)], out0.at[pl.ds(s * _RPT, _RPT)])

    @pl.when(c == 1)
    def _():
        pltpu.sync_copy(acc_sh.at[pl.ds(s * _RPT, _RPT)], out1.at[pl.ds(s * _RPT, _RPT)])


@functools.partial(
    pl.kernel,
    out_type=(
        jax.ShapeDtypeStruct((_NPAD, _D), jnp.float32),
        jax.ShapeDtypeStruct((_NPAD, _D), jnp.float32),
    ),
    mesh=_mesh,
    scratch_types=[
        pltpu.VMEM((_GC, _CK), jnp.int32),
        pltpu.VMEM((_GC, _CK), jnp.int32),
        pltpu.VMEM((2, _CK, _D), jnp.float32),
        pltpu.VMEM_SHARED((_NPAD, _D), jnp.float32),
        pltpu.SemaphoreType.DMA,
        pltpu.SemaphoreType.DMA,
    ],
)
def _sc_seg(*refs):
    _sc_seg_body(*refs)


def _tc_prep_body(deg0_ref, deg1_ref, x_ref, dinv_ref, g_ref):
    d = deg0_ref[...] + deg1_ref[...] + 1.0
    dcol = lax.rsqrt(d)[:, None]
    dinv_ref[...] = dcol
    g_ref[...] = dcol * x_ref[...]


def _tc_prep(deg0, deg1, x_pad):
    grid = _NPAD // _BN
    return pl.pallas_call(
        _tc_prep_body,
        grid=(grid,),
        in_specs=[
            pl.BlockSpec((_BN,), lambda i: (i,)),
            pl.BlockSpec((_BN,), lambda i: (i,)),
            pl.BlockSpec((_BN, _D), lambda i: (i, 0)),
        ],
        out_specs=[
            pl.BlockSpec((_BN, 1), lambda i: (i, 0)),
            pl.BlockSpec((_BN, _D), lambda i: (i, 0)),
        ],
        out_shape=[
            jax.ShapeDtypeStruct((_NPAD, 1), jnp.float32),
            jax.ShapeDtypeStruct((_NPAD, _D), jnp.float32),
        ],
    )(deg0, deg1, x_pad)


def _tc_mid_body(acc0_ref, acc1_ref, g_ref, dinv_ref, w_ref, b_ref, out_ref):
    dcol = dinv_ref[...]
    agg = dcol * (acc0_ref[...] + acc1_ref[...] + g_ref[...])
    h = jnp.dot(agg, w_ref[...], preferred_element_type=jnp.float32) + b_ref[...]
    out_ref[...] = dcol * jnp.maximum(h, 0.0)


def _tc_mid(acc0, acc1, g, dinv, w1t, b1r):
    grid = _NPAD // _BN
    return pl.pallas_call(
        _tc_mid_body,
        grid=(grid,),
        in_specs=[
            pl.BlockSpec((_BN, _D), lambda i: (i, 0)),
            pl.BlockSpec((_BN, _D), lambda i: (i, 0)),
            pl.BlockSpec((_BN, _D), lambda i: (i, 0)),
            pl.BlockSpec((_BN, 1), lambda i: (i, 0)),
            pl.BlockSpec((_D, _D), lambda i: (0, 0)),
            pl.BlockSpec((1, _D), lambda i: (0, 0)),
        ],
        out_specs=pl.BlockSpec((_BN, _D), lambda i: (i, 0)),
        out_shape=jax.ShapeDtypeStruct((_NPAD, _D), jnp.float32),
    )(acc0, acc1, g, dinv, w1t, b1r)


def _tc_final_body(acc0_ref, acc1_ref, g_ref, dinv_ref, w2_ref, b2_ref, w3_ref, b3_ref, out_ref):
    agg = dinv_ref[...] * (acc0_ref[...] + acc1_ref[...] + g_ref[...])
    h = jnp.dot(agg, w2_ref[...], preferred_element_type=jnp.float32) + b2_ref[...]
    h = jnp.maximum(h, 0.0)
    out_ref[...] = jnp.dot(h, w3_ref[...], preferred_element_type=jnp.float32) + b3_ref[...]


def _tc_final(acc0, acc1, g, dinv, w2t, b2r, w3t, b3r):
    grid = _NPAD // _BN
    cpad = w3t.shape[1]
    return pl.pallas_call(
        _tc_final_body,
        grid=(grid,),
        in_specs=[
            pl.BlockSpec((_BN, _D), lambda i: (i, 0)),
            pl.BlockSpec((_BN, _D), lambda i: (i, 0)),
            pl.BlockSpec((_BN, _D), lambda i: (i, 0)),
            pl.BlockSpec((_BN, 1), lambda i: (i, 0)),
            pl.BlockSpec((_D, _D), lambda i: (0, 0)),
            pl.BlockSpec((1, _D), lambda i: (0, 0)),
            pl.BlockSpec((_D, cpad), lambda i: (0, 0)),
            pl.BlockSpec((1, cpad), lambda i: (0, 0)),
        ],
        out_specs=pl.BlockSpec((_BN, cpad), lambda i: (i, 0)),
        out_shape=jax.ShapeDtypeStruct((_NPAD, cpad), jnp.float32),
    )(acc0, acc1, g, dinv, w2t, b2r, w3t, b3r)


def kernel(x, edge_index, W1, b1, W2, b2, W3, b3):
    n, d = x.shape
    e = edge_index.shape[1]
    cpad = 48

    x_pad = jnp.zeros((_NPAD, _D), jnp.float32).at[:n].set(x)
    src = jnp.full((_EPAD,), _N, jnp.int32).at[:e].set(edge_index[0])
    dst = jnp.full((_EPAD,), _N, jnp.int32).at[:e].set(edge_index[1])
    src2d = src.reshape(_W * _CW, _CK)
    dst2d = dst.reshape(_W * _CW, _CK)

    w1t = W1.T
    w2t = W2.T
    w3t = jnp.zeros((_D, cpad), jnp.float32).at[:, : W3.shape[0]].set(W3.T)
    b1r = b1.reshape(1, _D)
    b2r = b2.reshape(1, _D)
    b3r = jnp.zeros((1, cpad), jnp.float32).at[0, : W3.shape[0]].set(b3)

    deg0, deg1 = _sc_deg(dst2d)
    dinv, g1 = _tc_prep(deg0, deg1, x_pad)
    a10, a11 = _sc_seg(g1, src2d, dst2d)
    g2 = _tc_mid(a10, a11, g1, dinv, w1t, b1r)
    a20, a21 = _sc_seg(g2, src2d, dst2d)
    outp = _tc_final(a20, a21, g2, dinv, w2t, b2r, w3t, b3r)
    return outp[:n, : W3.shape[0]]


# speedup vs baseline: 1.2118x; 1.0167x over previous
import functools

import jax
import jax.numpy as jnp
from jax import lax
from jax.experimental import pallas as pl
from jax.experimental.pallas import tpu as pltpu
from jax.experimental.pallas import tpu_sc as plsc

_N = 10000
_D = 128
_NPAD = 10240
_NC = 2
_NS = 16
_W = _NC * _NS
_RPT = _NPAD // _NS
_CK = 128
_CW = 80
_GC = 40
_EPAD = _W * _CW * _CK
_BN = 1024

_mesh = plsc.VectorSubcoreMesh(core_axis_name="c", subcore_axis_name="s")


def _sc_deg_body(dst2d, deg0, deg1, idx_v, ones_v, zero_v, deg_sh, sem):
    c = lax.axis_index("c")
    s = lax.axis_index("s")
    wid = c * _NS + s

    def fill_ones(i, carry):
        ones_v[pl.ds(i * 16, 16)] = jnp.full((16,), 1.0, jnp.float32)
        return carry

    lax.fori_loop(0, _CK // 16, fill_ones, 0)

    def fill_zero(i, carry):
        zero_v[pl.ds(i * 16, 16)] = jnp.zeros((16,), jnp.float32)
        return carry

    lax.fori_loop(0, 640 // 16, fill_zero, 0)

    pltpu.sync_copy(zero_v.at[pl.ds(0, _RPT)], deg_sh.at[pl.ds(s * _RPT, _RPT)])
    pltpu.sync_copy(dst2d.at[pl.ds(wid * _CW, _CW)], idx_v)
    plsc.subcore_barrier()

    def body(j, carry):
        pltpu.sync_copy(ones_v, deg_sh.at[idx_v.at[j]], add=True)
        return carry

    lax.fori_loop(0, _CW, body, 0)
    plsc.subcore_barrier()

    pltpu.sync_copy(deg_sh.at[pl.ds(s * _RPT, _RPT)], zero_v.at[pl.ds(0, _RPT)])

    @pl.when(c == 0)
    def _():
        pltpu.sync_copy(zero_v.at[pl.ds(0, _RPT)], deg0.at[pl.ds(s * _RPT, _RPT)])

    @pl.when(c == 1)
    def _():
        pltpu.sync_copy(zero_v.at[pl.ds(0, _RPT)], deg1.at[pl.ds(s * _RPT, _RPT)])


@functools.partial(
    pl.kernel,
    out_type=(
        jax.ShapeDtypeStruct((_NPAD,), jnp.float32),
        jax.ShapeDtypeStruct((_NPAD,), jnp.float32),
    ),
    mesh=_mesh,
    scratch_types=[
        pltpu.VMEM((_CW, _CK), jnp.int32),
        pltpu.VMEM((_CK,), jnp.float32),
        pltpu.VMEM((640,), jnp.float32),
        pltpu.VMEM_SHARED((_NPAD,), jnp.float32),
        pltpu.SemaphoreType.DMA,
    ],
)
def _sc_deg(*refs):
    _sc_deg_body(*refs)


def _sc_seg_body(g_hbm, src2d, dst2d, out0, out1, sidx, didx, rows2, acc_sh, sem0, sem1):
    c = lax.axis_index("c")
    s = lax.axis_index("s")
    wid = c * _NS + s
    buf0 = rows2.at[0]
    buf1 = rows2.at[1]
    dummy = g_hbm.at[pl.ds(0, _CK)]

    def fz(i, carry):
        rows2[0, i // 8, pl.ds((i % 8) * 16, 16)] = jnp.zeros((16,), jnp.float32)
        return carry

    lax.fori_loop(0, _CK * 8, fz, 0)
    for q in range(_RPT // _CK):
        pltpu.sync_copy(buf0, acc_sh.at[pl.ds(s * _RPT + q * _CK, _CK)])
    plsc.subcore_barrier()

    for g in range(_CW // _GC):
        base = wid * _CW + g * _GC
        pltpu.sync_copy(src2d.at[pl.ds(base, _GC)], sidx)
        pltpu.sync_copy(dst2d.at[pl.ds(base, _GC)], didx)
        pltpu.async_copy(g_hbm.at[sidx.at[0]], buf0, sem0)

        def body(t, carry):
            pltpu.async_copy(g_hbm.at[sidx.at[2 * t + 1]], buf1, sem1)
            pltpu.make_async_copy(dummy, buf0, sem0).wait()
            pltpu.sync_copy(buf0, acc_sh.at[didx.at[2 * t]], add=True)

            @pl.when(t < _GC // 2 - 1)
            def _():
                pltpu.async_copy(g_hbm.at[sidx.at[2 * t + 2]], buf0, sem0)

            pltpu.make_async_copy(dummy, buf1, sem1).wait()
            pltpu.sync_copy(buf1, acc_sh.at[didx.at[2 * t + 1]], add=True)
            return carry

        lax.fori_loop(0, _GC // 2, body, 0)
    plsc.subcore_barrier()

    @pl.when(c == 0)
    def _():
        pltpu.sync_copy(acc_sh.at[pl.ds(s * _RPT, _RPT)], out0.at[pl.ds(s * _RPT, _RPT)])

    @pl.when(c == 1)
    def _():
        pltpu.sync_copy(acc_sh.at[pl.ds(s * _RPT, _RPT)], out1.at[pl.ds(s * _RPT, _RPT)])


@functools.partial(
    pl.kernel,
    out_type=(
        jax.ShapeDtypeStruct((_NPAD, _D), jnp.float32),
        jax.ShapeDtypeStruct((_NPAD, _D), jnp.float32),
    ),
    mesh=_mesh,
    scratch_types=[
        pltpu.VMEM((_GC, _CK), jnp.int32),
        pltpu.VMEM((_GC, _CK), jnp.int32),
        pltpu.VMEM((2, _CK, _D), jnp.float32),
        pltpu.VMEM_SHARED((_NPAD, _D), jnp.float32),
        pltpu.SemaphoreType.DMA,
        pltpu.SemaphoreType.DMA,
    ],
)
def _sc_seg(*refs):
    _sc_seg_body(*refs)


def _tc_prep_body(deg0_ref, deg1_ref, x_ref, dinv_ref, g_ref):
    d = deg0_ref[...] + deg1_ref[...] + 1.0
    dcol = lax.rsqrt(d)[:, None]
    dinv_ref[...] = dcol
    g_ref[...] = dcol * x_ref[...]


def _tc_prep(deg0, deg1, x_pad):
    grid = _NPAD // _BN
    return pl.pallas_call(
        _tc_prep_body,
        grid=(grid,),
        in_specs=[
            pl.BlockSpec((_BN,), lambda i: (i,)),
            pl.BlockSpec((_BN,), lambda i: (i,)),
            pl.BlockSpec((_BN, _D), lambda i: (i, 0)),
        ],
        out_specs=[
            pl.BlockSpec((_BN, 1), lambda i: (i, 0)),
            pl.BlockSpec((_BN, _D), lambda i: (i, 0)),
        ],
        out_shape=[
            jax.ShapeDtypeStruct((_NPAD, 1), jnp.float32),
            jax.ShapeDtypeStruct((_NPAD, _D), jnp.float32),
        ],
    )(deg0, deg1, x_pad)


def _tc_mid_body(acc0_ref, acc1_ref, g_ref, dinv_ref, w_ref, b_ref, out_ref):
    dcol = dinv_ref[...]
    agg = dcol * (acc0_ref[...] + acc1_ref[...] + g_ref[...])
    h = jnp.dot(agg, w_ref[...], preferred_element_type=jnp.float32) + b_ref[...]
    out_ref[...] = dcol * jnp.maximum(h, 0.0)


def _tc_mid(acc0, acc1, g, dinv, w1t, b1r):
    grid = _NPAD // _BN
    return pl.pallas_call(
        _tc_mid_body,
        grid=(grid,),
        in_specs=[
            pl.BlockSpec((_BN, _D), lambda i: (i, 0)),
            pl.BlockSpec((_BN, _D), lambda i: (i, 0)),
            pl.BlockSpec((_BN, _D), lambda i: (i, 0)),
            pl.BlockSpec((_BN, 1), lambda i: (i, 0)),
            pl.BlockSpec((_D, _D), lambda i: (0, 0)),
            pl.BlockSpec((1, _D), lambda i: (0, 0)),
        ],
        out_specs=pl.BlockSpec((_BN, _D), lambda i: (i, 0)),
        out_shape=jax.ShapeDtypeStruct((_NPAD, _D), jnp.float32),
    )(acc0, acc1, g, dinv, w1t, b1r)


def _tc_final_body(acc0_ref, acc1_ref, g_ref, dinv_ref, w2_ref, b2_ref, w3_ref, b3_ref, out_ref):
    agg = dinv_ref[...] * (acc0_ref[...] + acc1_ref[...] + g_ref[...])
    h = jnp.dot(agg, w2_ref[...], preferred_element_type=jnp.float32) + b2_ref[...]
    h = jnp.maximum(h, 0.0)
    out_ref[...] = jnp.dot(h, w3_ref[...], preferred_element_type=jnp.float32) + b3_ref[...]


def _tc_final(acc0, acc1, g, dinv, w2t, b2r, w3t, b3r):
    grid = _NPAD // _BN
    cpad = w3t.shape[1]
    return pl.pallas_call(
        _tc_final_body,
        grid=(grid,),
        in_specs=[
            pl.BlockSpec((_BN, _D), lambda i: (i, 0)),
            pl.BlockSpec((_BN, _D), lambda i: (i, 0)),
            pl.BlockSpec((_BN, _D), lambda i: (i, 0)),
            pl.BlockSpec((_BN, 1), lambda i: (i, 0)),
            pl.BlockSpec((_D, _D), lambda i: (0, 0)),
            pl.BlockSpec((1, _D), lambda i: (0, 0)),
            pl.BlockSpec((_D, cpad), lambda i: (0, 0)),
            pl.BlockSpec((1, cpad), lambda i: (0, 0)),
        ],
        out_specs=pl.BlockSpec((_BN, cpad), lambda i: (i, 0)),
        out_shape=jax.ShapeDtypeStruct((_NPAD, cpad), jnp.float32),
    )(acc0, acc1, g, dinv, w2t, b2r, w3t, b3r)


def kernel(x, edge_index, W1, b1, W2, b2, W3, b3):
    n, d = x.shape
    e = edge_index.shape[1]
    cpad = 48

    x_pad = jnp.zeros((_NPAD, _D), jnp.float32).at[:n].set(x)
    src = jnp.full((_EPAD,), _N, jnp.int32).at[:e].set(edge_index[0])
    dst = jnp.full((_EPAD,), _N, jnp.int32).at[:e].set(edge_index[1])
    src2d = src.reshape(_W * _CW, _CK)
    dst2d = dst.reshape(_W * _CW, _CK)

    w1t = W1.T
    w2t = W2.T
    w3t = jnp.zeros((_D, cpad), jnp.float32).at[:, : W3.shape[0]].set(W3.T)
    b1r = b1.reshape(1, _D)
    b2r = b2.reshape(1, _D)
    b3r = jnp.zeros((1, cpad), jnp.float32).at[0, : W3.shape[0]].set(b3)

    deg0, deg1 = _sc_deg(dst2d)
    dinv, g1 = _tc_prep(deg0, deg1, x_pad)
    a10, a11 = _sc_seg(g1, src2d, dst2d)
    g2 = _tc_mid(a10, a11, g1, dinv, w1t, b1r)
    a20, a21 = _sc_seg(g2, src2d, dst2d)
    outp = _tc_final(a20, a21, g2, dinv, w2t, b2r, w3t, b3r)
    return outp[:n, : W3.shape[0]]


# speedup vs baseline: 3.8326x; 3.1628x over previous
import functools

import jax
import jax.numpy as jnp
from jax import lax
from jax.experimental import pallas as pl
from jax.experimental.pallas import tpu as pltpu
from jax.experimental.pallas import tpu_sc as plsc

_N = 10000
_D = 128
_NPAD = 10240
_NC = 2
_NS = 16
_W = _NC * _NS
_RPT = _NPAD // _NS
_CK = 128
_CW = 80
_GC = 40
_EPAD = _W * _CW * _CK
_BN = 1024

_mesh = plsc.VectorSubcoreMesh(core_axis_name="c", subcore_axis_name="s")


def _sc_deg_body(dst2d, deg0, deg1, idx_v, ones_v, zero_v, deg_sh, sem):
    c = lax.axis_index("c")
    s = lax.axis_index("s")
    wid = c * _NS + s

    def fill_ones(i, carry):
        ones_v[pl.ds(i * 16, 16)] = jnp.full((16,), 1.0, jnp.float32)
        return carry

    lax.fori_loop(0, _CK // 16, fill_ones, 0)

    def fill_zero(i, carry):
        zero_v[pl.ds(i * 16, 16)] = jnp.zeros((16,), jnp.float32)
        return carry

    lax.fori_loop(0, 640 // 16, fill_zero, 0)

    pltpu.sync_copy(zero_v.at[pl.ds(0, _RPT)], deg_sh.at[pl.ds(s * _RPT, _RPT)])
    pltpu.sync_copy(dst2d.at[pl.ds(wid * _CW, _CW)], idx_v)
    plsc.subcore_barrier()

    def body(j, carry):
        pltpu.sync_copy(ones_v, deg_sh.at[idx_v.at[j]], add=True)
        return carry

    lax.fori_loop(0, _CW, body, 0)
    plsc.subcore_barrier()

    pltpu.sync_copy(deg_sh.at[pl.ds(s * _RPT, _RPT)], zero_v.at[pl.ds(0, _RPT)])

    @pl.when(c == 0)
    def _():
        pltpu.sync_copy(zero_v.at[pl.ds(0, _RPT)], deg0.at[pl.ds(s * _RPT, _RPT)])

    @pl.when(c == 1)
    def _():
        pltpu.sync_copy(zero_v.at[pl.ds(0, _RPT)], deg1.at[pl.ds(s * _RPT, _RPT)])


@functools.partial(
    pl.kernel,
    out_type=(
        jax.ShapeDtypeStruct((_NPAD,), jnp.float32),
        jax.ShapeDtypeStruct((_NPAD,), jnp.float32),
    ),
    mesh=_mesh,
    scratch_types=[
        pltpu.VMEM((_CW, _CK), jnp.int32),
        pltpu.VMEM((_CK,), jnp.float32),
        pltpu.VMEM((640,), jnp.float32),
        pltpu.VMEM_SHARED((_NPAD,), jnp.float32),
        pltpu.SemaphoreType.DMA,
    ],
)
def _sc_deg(*refs):
    _sc_deg_body(*refs)


def _sc_seg_body(g_hbm, src2d, dst2d, out0, out1, sidx, didx, rows2, acc_sh, sem0, sem1):
    c = lax.axis_index("c")
    s = lax.axis_index("s")
    wid = c * _NS + s
    buf0 = rows2.at[0]
    buf1 = rows2.at[1]
    dummy = g_hbm.at[pl.ds(0, _CK)]

    def fz(i, carry):
        rows2[0, i // 8, pl.ds((i % 8) * 16, 16)] = jnp.zeros((16,), jnp.float32)
        return carry

    lax.fori_loop(0, _CK * 8, fz, 0)
    for q in range(_RPT // _CK):
        pltpu.sync_copy(buf0, acc_sh.at[pl.ds(s * _RPT + q * _CK, _CK)])
    plsc.subcore_barrier()

    for g in range(_CW // _GC):
        base = wid * _CW + g * _GC
        pltpu.sync_copy(src2d.at[pl.ds(base, _GC)], sidx)
        pltpu.sync_copy(dst2d.at[pl.ds(base, _GC)], didx)
        pltpu.async_copy(g_hbm.at[sidx.at[0]], buf0, sem0)

        def body(t, carry):
            pltpu.async_copy(g_hbm.at[sidx.at[2 * t + 1]], buf1, sem1)
            pltpu.make_async_copy(dummy, buf0, sem0).wait()
            pltpu.sync_copy(buf0, acc_sh.at[didx.at[2 * t]], add=True)

            @pl.when(t < _GC // 2 - 1)
            def _():
                pltpu.async_copy(g_hbm.at[sidx.at[2 * t + 2]], buf0, sem0)

            pltpu.make_async_copy(dummy, buf1, sem1).wait()
            pltpu.sync_copy(buf1, acc_sh.at[didx.at[2 * t + 1]], add=True)
            return carry

        lax.fori_loop(0, _GC // 2, body, 0)
    plsc.subcore_barrier()

    @pl.when(c == 0)
    def _():
        pltpu.sync_copy(acc_sh.at[pl.ds(s * _RPT, _RPT)], out0.at[pl.ds(s * _RPT, _RPT)])

    @pl.when(c == 1)
    def _():
        pltpu.sync_copy(acc_sh.at[pl.ds(s * _RPT, _RPT)], out1.at[pl.ds(s * _RPT, _RPT)])


@functools.partial(
    pl.kernel,
    out_type=(
        jax.ShapeDtypeStruct((_NPAD, _D), jnp.float32),
        jax.ShapeDtypeStruct((_NPAD, _D), jnp.float32),
    ),
    mesh=_mesh,
    scratch_types=[
        pltpu.VMEM((_GC, _CK), jnp.int32),
        pltpu.VMEM((_GC, _CK), jnp.int32),
        pltpu.VMEM((2, _CK, _D), jnp.float32),
        pltpu.VMEM_SHARED((_NPAD, _D), jnp.float32),
        pltpu.SemaphoreType.DMA,
        pltpu.SemaphoreType.DMA,
    ],
)
def _sc_seg(*refs):
    _sc_seg_body(*refs)


def _tc_prep_body(deg0_ref, deg1_ref, x_ref, dinv_ref, g_ref):
    d = deg0_ref[...] + deg1_ref[...] + 1.0
    dcol = lax.rsqrt(d)[:, None]
    dinv_ref[...] = dcol
    g_ref[...] = dcol * x_ref[...]


def _tc_prep(deg0, deg1, x_pad):
    grid = _NPAD // _BN
    return pl.pallas_call(
        _tc_prep_body,
        grid=(grid,),
        in_specs=[
            pl.BlockSpec((_BN,), lambda i: (i,)),
            pl.BlockSpec((_BN,), lambda i: (i,)),
            pl.BlockSpec((_BN, _D), lambda i: (i, 0)),
        ],
        out_specs=[
            pl.BlockSpec((_BN, 1), lambda i: (i, 0)),
            pl.BlockSpec((_BN, _D), lambda i: (i, 0)),
        ],
        out_shape=[
            jax.ShapeDtypeStruct((_NPAD, 1), jnp.float32),
            jax.ShapeDtypeStruct((_NPAD, _D), jnp.float32),
        ],
    )(deg0, deg1, x_pad)


def _tc_mid_body(acc0_ref, acc1_ref, g_ref, dinv_ref, w_ref, b_ref, out_ref):
    dcol = dinv_ref[...]
    agg = dcol * (acc0_ref[...] + acc1_ref[...] + g_ref[...])
    h = jnp.dot(agg, w_ref[...], preferred_element_type=jnp.float32) + b_ref[...]
    out_ref[...] = dcol * jnp.maximum(h, 0.0)


def _tc_mid(acc0, acc1, g, dinv, w1t, b1r):
    grid = _NPAD // _BN
    return pl.pallas_call(
        _tc_mid_body,
        grid=(grid,),
        in_specs=[
            pl.BlockSpec((_BN, _D), lambda i: (i, 0)),
            pl.BlockSpec((_BN, _D), lambda i: (i, 0)),
            pl.BlockSpec((_BN, _D), lambda i: (i, 0)),
            pl.BlockSpec((_BN, 1), lambda i: (i, 0)),
            pl.BlockSpec((_D, _D), lambda i: (0, 0)),
            pl.BlockSpec((1, _D), lambda i: (0, 0)),
        ],
        out_specs=pl.BlockSpec((_BN, _D), lambda i: (i, 0)),
        out_shape=jax.ShapeDtypeStruct((_NPAD, _D), jnp.float32),
    )(acc0, acc1, g, dinv, w1t, b1r)


def _tc_final_body(acc0_ref, acc1_ref, g_ref, dinv_ref, w2_ref, b2_ref, w3_ref, b3_ref, out_ref):
    agg = dinv_ref[...] * (acc0_ref[...] + acc1_ref[...] + g_ref[...])
    h = jnp.dot(agg, w2_ref[...], preferred_element_type=jnp.float32) + b2_ref[...]
    h = jnp.maximum(h, 0.0)
    out_ref[...] = jnp.dot(h, w3_ref[...], preferred_element_type=jnp.float32) + b3_ref[...]


def _tc_final(acc0, acc1, g, dinv, w2t, b2r, w3t, b3r):
    grid = _NPAD // _BN
    cpad = w3t.shape[1]
    return pl.pallas_call(
        _tc_final_body,
        grid=(grid,),
        in_specs=[
            pl.BlockSpec((_BN, _D), lambda i: (i, 0)),
            pl.BlockSpec((_BN, _D), lambda i: (i, 0)),
            pl.BlockSpec((_BN, _D), lambda i: (i, 0)),
            pl.BlockSpec((_BN, 1), lambda i: (i, 0)),
            pl.BlockSpec((_D, _D), lambda i: (0, 0)),
            pl.BlockSpec((1, _D), lambda i: (0, 0)),
            pl.BlockSpec((_D, cpad), lambda i: (0, 0)),
            pl.BlockSpec((1, cpad), lambda i: (0, 0)),
        ],
        out_specs=pl.BlockSpec((_BN, cpad), lambda i: (i, 0)),
        out_shape=jax.ShapeDtypeStruct((_NPAD, cpad), jnp.float32),
    )(acc0, acc1, g, dinv, w2t, b2r, w3t, b3r)


def kernel(x, edge_index, W1, b1, W2, b2, W3, b3):
    n, d = x.shape
    e = edge_index.shape[1]
    cpad = 48

    x_pad = jnp.zeros((_NPAD, _D), jnp.float32).at[:n].set(x)
    pad = _N + (jnp.arange(_EPAD - e, dtype=jnp.int32) % (_NPAD - _N))
    src = jnp.concatenate([edge_index[0].astype(jnp.int32), pad])
    dst = jnp.concatenate([edge_index[1].astype(jnp.int32), pad])
    src2d = src.reshape(_W * _CW, _CK)
    dst2d = dst.reshape(_W * _CW, _CK)

    w1t = W1.T
    w2t = W2.T
    w3t = jnp.zeros((_D, cpad), jnp.float32).at[:, : W3.shape[0]].set(W3.T)
    b1r = b1.reshape(1, _D)
    b2r = b2.reshape(1, _D)
    b3r = jnp.zeros((1, cpad), jnp.float32).at[0, : W3.shape[0]].set(b3)

    deg0, deg1 = _sc_deg(dst2d)
    dinv, g1 = _tc_prep(deg0, deg1, x_pad)
    a10, a11 = _sc_seg(g1, src2d, dst2d)
    g2 = _tc_mid(a10, a11, g1, dinv, w1t, b1r)
    a20, a21 = _sc_seg(g2, src2d, dst2d)
    outp = _tc_final(a20, a21, g2, dinv, w2t, b2r, w3t, b3r)
    return outp[:n, : W3.shape[0]]
